# Initial kernel scaffold; baseline (speedup 1.0000x reference)
import jax
import jax.numpy as jnp
from jax.experimental import pallas as pl


def kernel(x, table):
    raise NotImplementedError("write your pallas kernel here")


# speedup vs baseline: 1.1302x; 1.1302x over previous
import functools

import jax
import jax.numpy as jnp
from jax import lax
from jax.experimental import pallas as pl
from jax.experimental.pallas import tpu as pltpu
from jax.experimental.pallas import tpu_sc as plsc

V = 1000000
H = 32
B = 16384
F = 26

NC, NS, L = 2, 16, 16
NW = NC * NS
BPW = B // NW
C = 16
NCHUNK = BPW // C
IPC = C * F
NG = 4
IPG = IPC // NG


def _sc_embed(x2, table):
    mesh = plsc.VectorSubcoreMesh(core_axis_name="c", subcore_axis_name="s")

    @functools.partial(
        pl.kernel,
        out_type=jax.ShapeDtypeStruct((B * H * F,), jnp.float32),
        mesh=mesh,
        compiler_params=pltpu.CompilerParams(
            needs_layout_passes=False, use_tc_tiling_on_sc=False
        ),
        scratch_types=[
            pltpu.VMEM((IPC,), jnp.int32),
            pltpu.VMEM((IPC, H), jnp.float32),
            pltpu.VMEM((C * H * F,), jnp.float32),
            pltpu.SemaphoreType.DMA,
        ],
    )
    def body(x_hbm, table_hbm, out_hbm, idx_v, rows_v, out_v, sem):
        cid = lax.axis_index("c")
        sid = lax.axis_index("s")
        wid = sid * NC + cid
        iota = lax.iota(jnp.int32, L)

        def chunk(i, _):
            b0 = wid * BPW + i * C
            pltpu.sync_copy(x_hbm.at[pl.ds(b0 * F, IPC)], idx_v)
            cps = [
                pltpu.async_copy(
                    table_hbm.at[idx_v.at[pl.ds(g * IPG, IPG)]],
                    rows_v.at[pl.ds(g * IPG, IPG)],
                    sem,
                )
                for g in range(NG)
            ]
            for cp in cps:
                cp.wait()

            def trans(b, _):
                for f in range(F):
                    for half in range(H // L):
                        vec = rows_v[b * F + f, pl.ds(half * L, L)]
                        tgt = iota * F + (b * (H * F) + half * L * F + f)
                        plsc.store_scatter(out_v, [tgt], vec)
                return _

            lax.fori_loop(0, C, trans, None)
            pltpu.sync_copy(out_v, out_hbm.at[pl.ds(b0 * H * F, C * H * F)])
            return _

        lax.fori_loop(0, NCHUNK, chunk, None)

    return body(x2, table)


def kernel(x, table):
    x2 = x.astype(jnp.int32).reshape(B * F)
    out = _sc_embed(x2, table)
    return out.reshape(B, H, F)


# speedup vs baseline: 1.5457x; 1.3677x over previous
import functools

import jax
import jax.numpy as jnp
from jax import lax
from jax.experimental import pallas as pl
from jax.experimental.pallas import tpu as pltpu
from jax.experimental.pallas import tpu_sc as plsc

V = 1000000
H = 32
B = 16384
F = 26
XP = 128

NC, NS, L = 2, 16, 16
NW = NC * NS
TB = 128
TCW = B // TB // NW
FG = F // 2
NTC = B // TB


def _sc_embed(xt, table):
    mesh = plsc.VectorSubcoreMesh(core_axis_name="c", subcore_axis_name="s")

    @functools.partial(
        pl.kernel,
        out_type=jax.ShapeDtypeStruct((F, H // 8, NTC, 8 * TB), jnp.float32),
        mesh=mesh,
        compiler_params=pltpu.CompilerParams(
            needs_layout_passes=False, use_tc_tiling_on_sc=False
        ),
        scratch_types=[
            pltpu.VMEM((FG, TB), jnp.int32),
            pltpu.VMEM((FG * TB, H), jnp.float32),
            pltpu.VMEM((FG, H // 8, 1, 8 * TB), jnp.float32),
            pltpu.SemaphoreType.DMA,
        ],
    )
    def body(x_hbm, table_hbm, out_hbm, idx_v, rows_v, out_v, sem):
        cid = lax.axis_index("c")
        sid = lax.axis_index("s")
        wid = sid * NC + cid
        iota = lax.iota(jnp.int32, L)
        zeros = iota * 0
        h8_0 = iota // 8
        h8_1 = (iota + L) // 8
        hrb_0 = (iota % 8) * TB
        hrb_1 = hrb_0

        def tile_col(it, _):
            k = it // 2
            fg = it % 2
            tc = wid * TCW + k
            b0 = tc * TB
            f0 = fg * FG
            pltpu.sync_copy(
                x_hbm.at[pl.ds(f0, FG), pl.ds(b0, TB)], idx_v
            )
            cps = [
                pltpu.async_copy(
                    table_hbm.at[idx_v.at[j]],
                    rows_v.at[pl.ds(j * TB, TB)],
                    sem,
                )
                for j in range(FG)
            ]
            for cp in cps:
                cp.wait()

            for j in range(FG):
                jvec = zeros + j

                def trans(b, _):
                    r = j * TB + b
                    v0 = rows_v[r, pl.ds(0, L)]
                    v1 = rows_v[r, pl.ds(L, L)]
                    plsc.store_scatter(
                        out_v, [jvec, h8_0, zeros, hrb_0 + b], v0
                    )
                    plsc.store_scatter(
                        out_v, [jvec, h8_1, zeros, hrb_1 + b], v1
                    )
                    return _

                lax.fori_loop(0, TB, trans, None, unroll=4)

            pltpu.sync_copy(
                out_v,
                out_hbm.at[pl.ds(f0, FG), :, pl.ds(tc, 1), :],
            )
            return _

        lax.fori_loop(0, 2 * TCW, tile_col, None)

    return body(xt, table)


def kernel(x, table):
    xt = jnp.transpose(x.astype(jnp.int32))
    out = _sc_embed(xt, table)
    o5 = out.reshape(F, H // 8, NTC, 8, TB)
    return o5.transpose(2, 4, 1, 3, 0).reshape(B, H, F)


# speedup vs baseline: 2.2417x; 1.4503x over previous
import functools

import jax
import jax.numpy as jnp
from jax import lax
from jax.experimental import pallas as pl
from jax.experimental.pallas import tpu as pltpu
from jax.experimental.pallas import tpu_sc as plsc

V = 1000000
VP = 4 * V
H = 32
B = 16384
F = 26
XP = 128

NC, NS, L = 2, 16, 16
NW = NC * NS
TB = 128
TCW = B // TB // NW
FG = F // 2
NTC = B // TB


TBLK = 8192


def _tc_table_relayout(tt):
    def body(x_ref, o_ref):
        o_ref[:, :H] = x_ref[...].T

    grid = (V + TBLK - 1) // TBLK
    return pl.pallas_call(
        body,
        grid=(grid,),
        in_specs=[pl.BlockSpec((H, TBLK), lambda i: (0, i))],
        out_specs=pl.BlockSpec((TBLK, 128), lambda i: (i, 0)),
        out_shape=jax.ShapeDtypeStruct((V, 128), jnp.float32),
    )(tt)


def _sc_embed(xt, table):
    mesh = plsc.VectorSubcoreMesh(core_axis_name="c", subcore_axis_name="s")

    @functools.partial(
        pl.kernel,
        out_type=jax.ShapeDtypeStruct((F, H // 8, NTC, 8 * TB), jnp.float32),
        mesh=mesh,
        compiler_params=pltpu.CompilerParams(
            needs_layout_passes=False, use_tc_tiling_on_sc=False
        ),
        scratch_types=[
            pltpu.VMEM((FG, TB), jnp.int32),
            pltpu.VMEM((FG * TB, H), jnp.float32),
            pltpu.VMEM((FG, H // 8, 1, 8 * TB), jnp.float32),
            pltpu.SemaphoreType.DMA,
        ],
    )
    def body(x_hbm, table_hbm, out_hbm, idx_v, rows_v, out_v, sem):
        cid = lax.axis_index("c")
        sid = lax.axis_index("s")
        wid = sid * NC + cid
        iota = lax.iota(jnp.int32, L)
        zeros = iota * 0
        h8_0 = iota // 8
        h8_1 = (iota + L) // 8
        hrb_0 = (iota % 8) * TB
        hrb_1 = hrb_0

        def tile_col(it, _):
            k = it // 2
            fg = it % 2
            tc = wid * TCW + k
            b0 = tc * TB
            f0 = fg * FG
            pltpu.sync_copy(
                x_hbm.at[pl.ds(f0, FG), pl.ds(b0, TB)], idx_v
            )
            cps = [
                pltpu.async_copy(
                    table_hbm.at[idx_v.at[j]],
                    rows_v.at[pl.ds(j * TB, TB)],
                    sem,
                )
                for j in range(FG)
            ]
            for cp in cps:
                cp.wait()

            for j in range(FG):
                jvec = zeros + j

                def trans(b, _):
                    r = j * TB + b
                    v0 = rows_v[r, pl.ds(0, L)]
                    v1 = rows_v[r, pl.ds(L, L)]
                    plsc.store_scatter(
                        out_v, [jvec, h8_0, zeros, hrb_0 + b], v0
                    )
                    plsc.store_scatter(
                        out_v, [jvec, h8_1, zeros, hrb_1 + b], v1
                    )
                    return _

                lax.fori_loop(0, TB, trans, None, unroll=4)

            pltpu.sync_copy(
                out_v,
                out_hbm.at[pl.ds(f0, FG), :, pl.ds(tc, 1), :],
            )
            return _

        lax.fori_loop(0, 2 * TCW, tile_col, None)

    return body(xt, table)


def kernel(x, table):
    xt = jnp.transpose(x.astype(jnp.int32)) * 4
    tp = _tc_table_relayout(table.T).reshape(VP, H)
    out = _sc_embed(xt, tp)
    o5 = out.reshape(F, H // 8, NTC, 8, TB)
    return o5.transpose(2, 4, 1, 3, 0).reshape(B, H, F)


# speedup vs baseline: 2.3656x; 1.0553x over previous
import functools

import jax
import jax.numpy as jnp
from jax import lax
from jax.experimental import pallas as pl
from jax.experimental.pallas import tpu as pltpu
from jax.experimental.pallas import tpu_sc as plsc

V = 1000000
VP = 4 * V
H = 32
B = 16384
F = 26

NC, NS, L = 2, 16, 16
NW = NC * NS
TB = 128
TCW = B // TB // NW
BPW = TB * TCW
NTC = B // TB

FG = 13
HB = 64
UN = TCW * 2 * 2

TBLK = 8192


def _tc_table_relayout(tt):
    def body(x_ref, o_ref):
        o_ref[:, :H] = x_ref[...].T

    grid = (V + TBLK - 1) // TBLK
    return pl.pallas_call(
        body,
        grid=(grid,),
        in_specs=[pl.BlockSpec((H, TBLK), lambda i: (0, i))],
        out_specs=pl.BlockSpec((TBLK, 128), lambda i: (i, 0)),
        out_shape=jax.ShapeDtypeStruct((V, 128), jnp.float32),
    )(tt)


def _sc_embed(xt, table):
    mesh = plsc.VectorSubcoreMesh(core_axis_name="c", subcore_axis_name="s")

    @functools.partial(
        pl.kernel,
        out_type=jax.ShapeDtypeStruct((F, H // 8, NTC, 8, TB), jnp.float32),
        mesh=mesh,
        compiler_params=pltpu.CompilerParams(
            needs_layout_passes=False, use_tc_tiling_on_sc=False
        ),
        scratch_types=[
            pltpu.VMEM((F, BPW), jnp.int32),
            pltpu.VMEM((2, FG * HB, H), jnp.float32),
            pltpu.VMEM((FG, H // 8, 1, 8, HB), jnp.float32),
            pltpu.SemaphoreType.DMA,
            pltpu.SemaphoreType.DMA,
        ],
    )
    def body(x_hbm, table_hbm, out_hbm, idx_v, rows_v, out_v, sem_g, sem_o):
        cid = lax.axis_index("c")
        sid = lax.axis_index("s")
        wid = sid * NC + cid
        iota = lax.iota(jnp.int32, L)
        zeros = iota * 0
        h8_0 = iota // 8
        h8_1 = (iota + L) // 8
        hr_0 = iota % 8
        hr_1 = hr_0

        pltpu.sync_copy(x_hbm.at[:, pl.ds(wid * BPW, BPW)], idx_v)

        def fire(u, buf):
            boff = (u >> 2) * TB + (u & 1) * HB
            g = (u >> 1) & 1
            for j in range(FG):
                pltpu.async_copy(
                    table_hbm.at[idx_v.at[g * FG + j, pl.ds(boff, HB)]],
                    rows_v.at[buf, pl.ds(j * HB, HB)],
                    sem_g,
                )

        def drain_rows(buf):
            pltpu.make_async_copy(
                table_hbm.at[pl.ds(0, FG * HB)], rows_v.at[buf], sem_g
            ).wait()

        def drain_out():
            pltpu.make_async_copy(
                out_hbm.at[pl.ds(0, FG), :, pl.ds(0, 1), :, pl.ds(0, HB)],
                out_v,
                sem_o,
            ).wait()

        fire(0, 0)

        def step(u, _):
            buf = u & 1

            @pl.when(u + 1 < UN)
            def _fire_next():
                fire(u + 1, 1 - buf)

            drain_rows(buf)

            @pl.when(u > 0)
            def _drain_prev_out():
                drain_out()

            def trans(b, _):
                for j in range(FG):
                    r = j * HB + b
                    v0 = rows_v[buf, r, pl.ds(0, L)]
                    v1 = rows_v[buf, r, pl.ds(L, L)]
                    jvec = zeros + j
                    plsc.store_scatter(
                        out_v, [jvec, h8_0, zeros, hr_0, zeros + b], v0
                    )
                    plsc.store_scatter(
                        out_v, [jvec, h8_1, zeros, hr_1, zeros + b], v1
                    )
                return _

            lax.fori_loop(0, HB, trans, None, unroll=2)
            tc = wid * TCW + (u >> 2)
            g = (u >> 1) & 1
            bh = u & 1
            pltpu.async_copy(
                out_v,
                out_hbm.at[
                    pl.ds(g * FG, FG), :, pl.ds(tc, 1), :, pl.ds(bh * HB, HB)
                ],
                sem_o,
            )
            return _

        lax.fori_loop(0, UN, step, None)
        drain_out()

    return body(xt, table)


def kernel(x, table):
    xt = jnp.transpose(x.astype(jnp.int32)) * 4
    tp = _tc_table_relayout(table.T).reshape(VP, H)
    out = _sc_embed(xt, tp)
    o5 = out
    return o5.transpose(2, 4, 1, 3, 0).reshape(B, H, F)
